# baseline (speedup 1.0000x reference)
import functools

import jax
import jax.numpy as jnp
from jax import lax
from jax.experimental import pallas as pl
from jax.experimental.pallas import tpu as pltpu
from jax.experimental.pallas import tpu_sc as plsc

DIM = 64
BATCH = 16384
REL_ROWS = 1000
NC = 2
NS = 16
NW = NC * NS
BPW = BATCH // NW
C = 64
NCH = BPW // C
G = C // 16


def _transe_body(bh, bt, br, ent2, rel1d, out_hbm,
                 idx_h, idx_t, idx_r, idxh2, idxt2, hv, tv, relv, ov,
                 sem0, sem1):
    wid = lax.axis_index("s") * NC + lax.axis_index("c")
    base = wid * BPW
    sems = (sem0, sem1)

    pltpu.sync_copy(bh.at[pl.ds(base, BPW)], idx_h)
    pltpu.sync_copy(bt.at[pl.ds(base, BPW)], idx_t)
    pltpu.sync_copy(br.at[pl.ds(base, BPW)], idx_r)
    pltpu.sync_copy(rel1d, relv)

    @pl.loop(0, BPW // 16)
    def _shift(g):
        s = pl.ds(g * 16, 16)
        ch = g // (C // 16)
        off = (g % (C // 16)) * 16
        idxh2[ch, pl.ds(off, 16)] = idx_h[s] >> 1
        idxt2[ch, pl.ds(off, 16)] = idx_t[s] >> 1

    lanes = lax.iota(jnp.int32, 16)

    def fire(ch, sem):
        b = ch & 1
        pltpu.async_copy(ent2.at[idxh2.at[ch]],
                         hv.at[pl.ds(b * C, C)], sem)
        pltpu.async_copy(ent2.at[idxt2.at[ch]],
                         tv.at[pl.ds(b * C, C)], sem)

    def drain(ch, sem):
        b = ch & 1
        pltpu.make_async_copy(ent2.at[pl.ds(0, C)],
                              hv.at[pl.ds(b * C, C)], sem).wait()
        pltpu.make_async_copy(ent2.at[pl.ds(0, C)],
                              tv.at[pl.ds(b * C, C)], sem).wait()

    def compute(ch):
        b = ch & 1

        def group_body(g, _):
            s = pl.ds(ch * C + g * 16, 16)
            rows = b * C + g * 16 + lanes
            ph = (idx_h[s] & 1) * DIM
            pt = (idx_t[s] & 1) * DIM
            jr = idx_r[s] * DIM

            def col_body(j, acc):
                hg = plsc.load_gather(hv, [rows, ph + j])
                tg = plsc.load_gather(tv, [rows, pt + j])
                rg = plsc.load_gather(relv, [jr + j])
                return acc + jnp.abs(hg + rg - tg)

            acc = lax.fori_loop(0, DIM, col_body, jnp.zeros((16,), jnp.float32))
            ov[pl.ds(ch * C + g * 16, 16)] = acc
            return 0

        lax.fori_loop(0, G, group_body, 0)

    fire(0, sems[0])
    for ch in range(NCH):
        if ch + 1 < NCH:
            fire(ch + 1, sems[(ch + 1) & 1])
        drain(ch, sems[ch & 1])
        compute(ch)

    pltpu.sync_copy(ov, out_hbm.at[pl.ds(base, BPW)])


_transe = functools.partial(
    pl.kernel,
    out_type=jax.ShapeDtypeStruct((BATCH,), jnp.float32),
    mesh=plsc.VectorSubcoreMesh(core_axis_name="c", subcore_axis_name="s"),
    scratch_types=[
        pltpu.VMEM((BPW,), jnp.int32),
        pltpu.VMEM((BPW,), jnp.int32),
        pltpu.VMEM((BPW,), jnp.int32),
        pltpu.VMEM((NCH, C), jnp.int32),
        pltpu.VMEM((NCH, C), jnp.int32),
        pltpu.VMEM((2 * C, 2 * DIM), jnp.float32),
        pltpu.VMEM((2 * C, 2 * DIM), jnp.float32),
        pltpu.VMEM((REL_ROWS * DIM,), jnp.float32),
        pltpu.VMEM((BPW,), jnp.float32),
        pltpu.SemaphoreType.DMA,
        pltpu.SemaphoreType.DMA,
    ],
    compiler_params=pltpu.CompilerParams(needs_layout_passes=False),
)(_transe_body)


@jax.jit
def kernel(batch_h, batch_t, batch_r, ent_emb, rel_emb):
    ent2 = ent_emb.reshape(ent_emb.shape[0] // 2, 2 * DIM)
    return _transe(batch_h, batch_t, batch_r, ent2, rel_emb.reshape(-1))


# speedup vs baseline: 1.0037x; 1.0037x over previous
import functools

import jax
import jax.numpy as jnp
from jax import lax
from jax.experimental import pallas as pl
from jax.experimental.pallas import tpu as pltpu
from jax.experimental.pallas import tpu_sc as plsc

DIM = 64
BATCH = 16384
REL_ROWS = 1000
NC = 2
NS = 16
NW = NC * NS
BPW = BATCH // NW
C = 64
NCH = BPW // C
G = C // 16


def _transe_body(bh, bt, br, ent, rel1d, out_hbm,
                 idx_h, idx_t, idx_r, hv, tv, relv, ov, sem0, sem1):
    wid = lax.axis_index("s") * NC + lax.axis_index("c")
    sems = (sem0, sem1)

    pltpu.sync_copy(bh.at[pl.ds(wid * NCH, NCH)], idx_h)
    pltpu.sync_copy(bt.at[pl.ds(wid * NCH, NCH)], idx_t)
    pltpu.sync_copy(br.at[pl.ds(wid * NCH, NCH)], idx_r)
    pltpu.sync_copy(rel1d, relv)

    lanes = lax.iota(jnp.int32, 16)

    def fire(ch, sem):
        b = ch & 1
        pltpu.async_copy(ent.at[idx_h.at[ch]], hv.at[pl.ds(b * C, C)], sem)
        pltpu.async_copy(ent.at[idx_t.at[ch]], tv.at[pl.ds(b * C, C)], sem)

    def drain(ch, sem):
        b = ch & 1
        pltpu.make_async_copy(ent.at[pl.ds(0, C)],
                              hv.at[pl.ds(b * C, C)], sem).wait()
        pltpu.make_async_copy(ent.at[pl.ds(0, C)],
                              tv.at[pl.ds(b * C, C)], sem).wait()

    def compute(ch):
        b = ch & 1

        def group_body(g, _):
            rows = b * C + g * 16 + lanes
            jr = idx_r[ch, pl.ds(g * 16, 16)] * DIM

            def col_body(j, acc):
                colj = jnp.full((16,), 0, jnp.int32) + j
                hg = plsc.load_gather(hv, [rows, colj])
                tg = plsc.load_gather(tv, [rows, colj])
                rg = plsc.load_gather(relv, [jr + j])
                return acc + jnp.abs(hg + rg - tg)

            acc = lax.fori_loop(0, DIM, col_body, jnp.zeros((16,), jnp.float32))
            ov[pl.ds(ch * C + g * 16, 16)] = acc
            return 0

        lax.fori_loop(0, G, group_body, 0)

    fire(0, sems[0])
    for ch in range(NCH):
        if ch + 1 < NCH:
            fire(ch + 1, sems[(ch + 1) & 1])
        drain(ch, sems[ch & 1])
        compute(ch)

    pltpu.sync_copy(ov, out_hbm.at[pl.ds(wid * BPW, BPW)])


_transe = functools.partial(
    pl.kernel,
    out_type=jax.ShapeDtypeStruct((BATCH,), jnp.float32),
    mesh=plsc.VectorSubcoreMesh(core_axis_name="c", subcore_axis_name="s"),
    scratch_types=[
        pltpu.VMEM((NCH, C), jnp.int32),
        pltpu.VMEM((NCH, C), jnp.int32),
        pltpu.VMEM((NCH, C), jnp.int32),
        pltpu.VMEM((2 * C, DIM), jnp.float32),
        pltpu.VMEM((2 * C, DIM), jnp.float32),
        pltpu.VMEM((REL_ROWS * DIM,), jnp.float32),
        pltpu.VMEM((BPW,), jnp.float32),
        pltpu.SemaphoreType.DMA,
        pltpu.SemaphoreType.DMA,
    ],
    compiler_params=pltpu.CompilerParams(
        use_tc_tiling_on_sc=False, needs_layout_passes=False),
)(_transe_body)


@jax.jit
def kernel(batch_h, batch_t, batch_r, ent_emb, rel_emb):
    bh = batch_h.reshape(NW * NCH, C)
    bt = batch_t.reshape(NW * NCH, C)
    br = batch_r.reshape(NW * NCH, C)
    return _transe(bh, bt, br, ent_emb, rel_emb.reshape(-1))
